# baseline (speedup 1.0000x reference)
import functools

import jax
import jax.numpy as jnp
from jax import lax
from jax.experimental import pallas as pl
from jax.experimental.pallas import tpu as pltpu
from jax.experimental.pallas import tpu_sc as plsc

N = 10000
E = 320000
D = 128
NC = 2
NS = 16
NW = NC * NS
N_PAD = 10240
CH = 128
NPH = 2
PHR = 40
NCH = NPH * PHR
EPW = NCH * CH
RPT = N_PAD // NS
TB = 1024


def _sc_aggregate(with_cnt: bool):
    mesh = plsc.VectorSubcoreMesh(core_axis_name="c", subcore_axis_name="s")
    out_type = [jax.ShapeDtypeStruct((NC, N_PAD, D), jnp.float32)]
    scratch = [
        pltpu.VMEM((PHR, CH), jnp.int32),
        pltpu.VMEM((PHR, CH), jnp.int32),
        pltpu.VMEM((2, CH, D), jnp.float32),
        pltpu.VMEM_SHARED((N_PAD, D), jnp.float32),
        pltpu.SemaphoreType.DMA,
        pltpu.SemaphoreType.DMA,
    ]
    if with_cnt:
        out_type.append(jax.ShapeDtypeStruct((NC, N_PAD), jnp.float32))
        scratch += [
            pltpu.VMEM((CH,), jnp.float32),
            pltpu.VMEM_SHARED((N_PAD,), jnp.float32),
            pltpu.SemaphoreType.DMA,
        ]

    def body(x_hbm, src_hbm, dst_hbm, z2_hbm, z1_hbm, o1_hbm,
             out_agg, *rest):
        if with_cnt:
            (out_cnt, src_v, dst_v, rows_v, acc_sh, sem_g, sem_s,
             ones_v, cnt_sh, sem_c) = rest
        else:
            src_v, dst_v, rows_v, acc_sh, sem_g, sem_s = rest
        cid = lax.axis_index("c")
        sid = lax.axis_index("s")
        wid = cid * NS + sid
        pltpu.sync_copy(z2_hbm, acc_sh.at[pl.ds(sid * RPT, RPT)])
        if with_cnt:
            pltpu.sync_copy(z1_hbm, cnt_sh.at[pl.ds(sid * RPT, RPT)])
            pltpu.sync_copy(o1_hbm, ones_v)
        plsc.subcore_barrier()

        for p in range(NPH):
            pltpu.sync_copy(src_hbm.at[wid, pl.ds(p * PHR, PHR)], src_v)
            pltpu.sync_copy(dst_hbm.at[wid, pl.ds(p * PHR, PHR)], dst_v)

            def step(j, carry):
                pltpu.async_copy(x_hbm.at[src_v.at[j]],
                                 rows_v.at[0], sem_g).wait()
                pltpu.sync_copy(rows_v.at[0], acc_sh.at[dst_v.at[j]],
                                add=True)
                if with_cnt:
                    pltpu.async_copy(ones_v, cnt_sh.at[dst_v.at[j]],
                                     sem_c, add=True)
                return carry

            lax.fori_loop(0, PHR, step, 0)
            if with_cnt:
                pltpu.make_async_copy(dst_hbm.at[wid, pl.ds(0, PHR)],
                                      src_v, sem_c).wait()
        plsc.subcore_barrier()
        pltpu.sync_copy(acc_sh.at[pl.ds(sid * RPT, RPT)],
                        out_agg.at[cid, pl.ds(sid * RPT, RPT)])
        if with_cnt:
            pltpu.sync_copy(cnt_sh.at[pl.ds(sid * RPT, RPT)],
                            out_cnt.at[cid, pl.ds(sid * RPT, RPT)])

    return pl.kernel(body, out_type=out_type, mesh=mesh,
                     scratch_types=scratch)


def _tc_layer(relu: bool):
    def body(a0, a1, c0, c1, x, wl, wr, b, o):
        asum = a0[0] + a1[0]
        cs = jnp.maximum(c0[0] + c1[0], 1.0)
        mean = asum / cs
        r = (jnp.dot(mean, wl[...], preferred_element_type=jnp.float32)
             + jnp.dot(x[...], wr[...], preferred_element_type=jnp.float32)
             + b[...])
        o[...] = jnp.maximum(r, 0.0) if relu else r

    grid = (N_PAD // TB,)
    return pl.pallas_call(
        body,
        grid=grid,
        in_specs=[
            pl.BlockSpec((1, TB, D), lambda i: (0, i, 0)),
            pl.BlockSpec((1, TB, D), lambda i: (1, i, 0)),
            pl.BlockSpec((1, TB, 1), lambda i: (0, i, 0)),
            pl.BlockSpec((1, TB, 1), lambda i: (1, i, 0)),
            pl.BlockSpec((TB, D), lambda i: (i, 0)),
            pl.BlockSpec((D, D), lambda i: (0, 0)),
            pl.BlockSpec((D, D), lambda i: (0, 0)),
            pl.BlockSpec((1, D), lambda i: (0, 0)),
        ],
        out_specs=pl.BlockSpec((TB, D), lambda i: (i, 0)),
        out_shape=jax.ShapeDtypeStruct((N_PAD, D), jnp.float32),
    )


_sc_agg_cnt = _sc_aggregate(with_cnt=True)
_sc_agg = _sc_aggregate(with_cnt=False)
_tc_layer1 = _tc_layer(relu=True)
_tc_layer2 = _tc_layer(relu=False)


def kernel(x, edge_index, W1l, b1, W1r, W2l, b2, W2r):
    xp = jnp.pad(x.astype(jnp.float32), ((0, N_PAD - N), (0, 0)))
    src = edge_index[0].astype(jnp.int32).reshape(NW, E // NW)
    dst = edge_index[1].astype(jnp.int32).reshape(NW, E // NW)
    pad = ((0, 0), (0, EPW - E // NW))
    src = jnp.pad(src, pad).reshape(NW, NCH, CH)
    dst = jnp.pad(dst, pad, constant_values=N_PAD - 1).reshape(NW, NCH, CH)
    z2 = jnp.zeros((RPT, D), jnp.float32)
    z1 = jnp.zeros((RPT,), jnp.float32)
    o1 = jnp.ones((CH,), jnp.float32)

    agg1, cnt = _sc_agg_cnt(xp, src, dst, z2, z1, o1)
    cntc = cnt.reshape(NC, N_PAD, 1)
    w1l_t = W1l.T
    w1r_t = W1r.T
    h = _tc_layer1(agg1, agg1, cntc, cntc, xp, w1l_t, w1r_t,
                   b1.reshape(1, D))
    (agg2,) = _sc_agg(h, src, dst, z2, z1, o1)
    out = _tc_layer2(agg2, agg2, cntc, cntc, h, W2l.T, W2r.T,
                     b2.reshape(1, D))
    return out[:N]


# speedup vs baseline: 2.2727x; 2.2727x over previous
import functools

import jax
import jax.numpy as jnp
from jax import lax
from jax.experimental import pallas as pl
from jax.experimental.pallas import tpu as pltpu
from jax.experimental.pallas import tpu_sc as plsc

N = 10000
E = 320000
D = 128
NC = 2
NS = 16
NW = NC * NS
N_PAD = 10240
CH = 128
NPH = 2
PHR = 40
NCH = NPH * PHR
EPW = NCH * CH
RPT = N_PAD // NS
TB = 1024


def _sc_aggregate(with_cnt: bool):
    mesh = plsc.VectorSubcoreMesh(core_axis_name="c", subcore_axis_name="s")
    out_type = [jax.ShapeDtypeStruct((NC, N_PAD, D), jnp.float32)]
    scratch = [
        pltpu.VMEM((PHR, CH), jnp.int32),
        pltpu.VMEM((PHR, CH), jnp.int32),
        pltpu.VMEM((2, CH, D), jnp.float32),
        pltpu.VMEM_SHARED((N_PAD, D), jnp.float32),
        pltpu.SemaphoreType.DMA,
        pltpu.SemaphoreType.DMA,
    ]
    if with_cnt:
        out_type.append(jax.ShapeDtypeStruct((NC, N_PAD), jnp.float32))
        scratch += [
            pltpu.VMEM((CH,), jnp.float32),
            pltpu.VMEM_SHARED((N_PAD,), jnp.float32),
            pltpu.SemaphoreType.DMA,
        ]

    def body(x_hbm, src_hbm, dst_hbm, z2_hbm, z1_hbm, o1_hbm,
             out_agg, *rest):
        if with_cnt:
            (out_cnt, src_v, dst_v, rows_v, acc_sh, sem_g, sem_s,
             ones_v, cnt_sh, sem_c) = rest
        else:
            src_v, dst_v, rows_v, acc_sh, sem_g, sem_s = rest
        cid = lax.axis_index("c")
        sid = lax.axis_index("s")
        wid = cid * NS + sid
        pltpu.sync_copy(z2_hbm, acc_sh.at[pl.ds(sid * RPT, RPT)])
        if with_cnt:
            pltpu.sync_copy(z1_hbm, cnt_sh.at[pl.ds(sid * RPT, RPT)])
            pltpu.sync_copy(o1_hbm, ones_v)
        plsc.subcore_barrier()

        for p in range(NPH):
            pltpu.sync_copy(src_hbm.at[wid, pl.ds(p * PHR, PHR)], src_v)
            pltpu.sync_copy(dst_hbm.at[wid, pl.ds(p * PHR, PHR)], dst_v)

            def step(j, carry):
                pltpu.async_copy(x_hbm.at[src_v.at[j]],
                                 rows_v.at[0], sem_g).wait()
                pltpu.sync_copy(rows_v.at[0], acc_sh.at[dst_v.at[j]],
                                add=True)
                if with_cnt:
                    pltpu.async_copy(ones_v, cnt_sh.at[dst_v.at[j]],
                                     sem_c, add=True)
                return carry

            lax.fori_loop(0, PHR, step, 0)
            if with_cnt:
                pltpu.make_async_copy(dst_hbm.at[wid, pl.ds(0, PHR)],
                                      src_v, sem_c).wait()
        plsc.subcore_barrier()
        pltpu.sync_copy(acc_sh.at[pl.ds(sid * RPT, RPT)],
                        out_agg.at[cid, pl.ds(sid * RPT, RPT)])
        if with_cnt:
            pltpu.sync_copy(cnt_sh.at[pl.ds(sid * RPT, RPT)],
                            out_cnt.at[cid, pl.ds(sid * RPT, RPT)])

    return pl.kernel(body, out_type=out_type, mesh=mesh,
                     scratch_types=scratch)


def _tc_layer(relu: bool):
    def body(a0, a1, c0, c1, x, wl, wr, b, o):
        asum = a0[0] + a1[0]
        cs = jnp.maximum(c0[0] + c1[0], 1.0)
        mean = asum / cs
        r = (jnp.dot(mean, wl[...], preferred_element_type=jnp.float32)
             + jnp.dot(x[...], wr[...], preferred_element_type=jnp.float32)
             + b[...])
        o[...] = jnp.maximum(r, 0.0) if relu else r

    grid = (N_PAD // TB,)
    return pl.pallas_call(
        body,
        grid=grid,
        in_specs=[
            pl.BlockSpec((1, TB, D), lambda i: (0, i, 0)),
            pl.BlockSpec((1, TB, D), lambda i: (1, i, 0)),
            pl.BlockSpec((1, TB, 1), lambda i: (0, i, 0)),
            pl.BlockSpec((1, TB, 1), lambda i: (1, i, 0)),
            pl.BlockSpec((TB, D), lambda i: (i, 0)),
            pl.BlockSpec((D, D), lambda i: (0, 0)),
            pl.BlockSpec((D, D), lambda i: (0, 0)),
            pl.BlockSpec((1, D), lambda i: (0, 0)),
        ],
        out_specs=pl.BlockSpec((TB, D), lambda i: (i, 0)),
        out_shape=jax.ShapeDtypeStruct((N_PAD, D), jnp.float32),
    )


_sc_agg_cnt = _sc_aggregate(with_cnt=True)
_sc_agg = _sc_aggregate(with_cnt=False)
_tc_layer1 = _tc_layer(relu=True)
_tc_layer2 = _tc_layer(relu=False)


def kernel(x, edge_index, W1l, b1, W1r, W2l, b2, W2r):
    xp = jnp.pad(x.astype(jnp.float32), ((0, N_PAD - N), (0, 0)))
    src = edge_index[0].astype(jnp.int32).reshape(NW, E // NW)
    dst = edge_index[1].astype(jnp.int32).reshape(NW, E // NW)
    padw = EPW - E // NW
    pad_src = jnp.broadcast_to(jnp.arange(padw, dtype=jnp.int32) % N,
                               (NW, padw))
    pad_dst = jnp.broadcast_to(
        N + jnp.arange(padw, dtype=jnp.int32) % (N_PAD - N), (NW, padw))
    src = jnp.concatenate([src, pad_src], axis=1).reshape(NW, NCH, CH)
    dst = jnp.concatenate([dst, pad_dst], axis=1).reshape(NW, NCH, CH)
    z2 = jnp.zeros((RPT, D), jnp.float32)
    z1 = jnp.zeros((RPT,), jnp.float32)
    o1 = jnp.ones((CH,), jnp.float32)

    agg1, cnt = _sc_agg_cnt(xp, src, dst, z2, z1, o1)
    cntc = cnt.reshape(NC, N_PAD, 1)
    w1l_t = W1l.T
    w1r_t = W1r.T
    h = _tc_layer1(agg1, agg1, cntc, cntc, xp, w1l_t, w1r_t,
                   b1.reshape(1, D))
    (agg2,) = _sc_agg(h, src, dst, z2, z1, o1)
    out = _tc_layer2(agg2, agg2, cntc, cntc, h, W2l.T, W2r.T,
                     b2.reshape(1, D))
    return out[:N]


# speedup vs baseline: 2.8716x; 1.2635x over previous
import functools

import jax
import jax.numpy as jnp
from jax import lax
from jax.experimental import pallas as pl
from jax.experimental.pallas import tpu as pltpu
from jax.experimental.pallas import tpu_sc as plsc

N = 10000
E = 320000
D = 128
NC = 2
NS = 16
NW = NC * NS
N_PAD = 10240
CH = 128
NPH = 2
PHR = 40
NCH = NPH * PHR
EPW = NCH * CH
RPT = N_PAD // NS
TB = 1024


def _sc_aggregate(with_cnt: bool):
    mesh = plsc.VectorSubcoreMesh(core_axis_name="c", subcore_axis_name="s")
    out_type = [jax.ShapeDtypeStruct((NC, N_PAD, D), jnp.float32)]
    scratch = [
        pltpu.VMEM((PHR, CH), jnp.int32),
        pltpu.VMEM((PHR, CH), jnp.int32),
        pltpu.VMEM((2, CH, D), jnp.float32),
        pltpu.VMEM_SHARED((N_PAD, D), jnp.float32),
        pltpu.SemaphoreType.DMA,
        pltpu.SemaphoreType.DMA,
    ]
    if with_cnt:
        out_type.append(jax.ShapeDtypeStruct((NC, N_PAD), jnp.float32))
        scratch += [
            pltpu.VMEM((CH,), jnp.float32),
            pltpu.VMEM_SHARED((N_PAD,), jnp.float32),
            pltpu.SemaphoreType.DMA,
        ]

    def body(x_hbm, src_hbm, dst_hbm, z2_hbm, z1_hbm, o1_hbm,
             out_agg, *rest):
        if with_cnt:
            (out_cnt, src_v, dst_v, rows_v, acc_sh, sem_g, sem_s,
             ones_v, cnt_sh, sem_c) = rest
        else:
            src_v, dst_v, rows_v, acc_sh, sem_g, sem_s = rest
        cid = lax.axis_index("c")
        sid = lax.axis_index("s")
        wid = cid * NS + sid
        pltpu.sync_copy(z2_hbm, acc_sh.at[pl.ds(sid * RPT, RPT)])
        if with_cnt:
            pltpu.sync_copy(z1_hbm, cnt_sh.at[pl.ds(sid * RPT, RPT)])
            pltpu.sync_copy(o1_hbm, ones_v)
        plsc.subcore_barrier()

        for p in range(NPH):
            pltpu.sync_copy(src_hbm.at[wid, pl.ds(p * PHR, PHR)], src_v)
            pltpu.sync_copy(dst_hbm.at[wid, pl.ds(p * PHR, PHR)], dst_v)
            pltpu.async_copy(x_hbm.at[src_v.at[0]], rows_v.at[0], sem_g)

            def step(j, carry):
                b = lax.rem(j, 2)
                nb = lax.rem(j + 1, 2)
                pltpu.make_async_copy(x_hbm.at[src_v.at[j]],
                                      rows_v.at[b], sem_g).wait()
                pltpu.async_copy(rows_v.at[b], acc_sh.at[dst_v.at[j]],
                                 sem_s, add=True)

                @pl.when(j + 1 < PHR)
                def _():
                    pltpu.async_copy(x_hbm.at[src_v.at[j + 1]],
                                     rows_v.at[nb], sem_g)

                if with_cnt:
                    pltpu.async_copy(ones_v, cnt_sh.at[dst_v.at[j]],
                                     sem_c, add=True)
                return carry

            lax.fori_loop(0, PHR, step, 0)

            def sdrain(j, carry):
                pltpu.make_async_copy(rows_v.at[0], acc_sh.at[dst_v.at[j]],
                                      sem_s).wait()
                return carry

            lax.fori_loop(0, PHR, sdrain, 0)
            if with_cnt:
                pltpu.make_async_copy(dst_hbm.at[wid, pl.ds(0, PHR)],
                                      src_v, sem_c).wait()
        plsc.subcore_barrier()
        pltpu.sync_copy(acc_sh.at[pl.ds(sid * RPT, RPT)],
                        out_agg.at[cid, pl.ds(sid * RPT, RPT)])
        if with_cnt:
            pltpu.sync_copy(cnt_sh.at[pl.ds(sid * RPT, RPT)],
                            out_cnt.at[cid, pl.ds(sid * RPT, RPT)])

    return pl.kernel(body, out_type=out_type, mesh=mesh,
                     scratch_types=scratch)


def _tc_layer(relu: bool):
    def body(a0, a1, c0, c1, x, wl, wr, b, o):
        asum = a0[0] + a1[0]
        cs = jnp.maximum(c0[0] + c1[0], 1.0)
        mean = asum / cs
        r = (jnp.dot(mean, wl[...], preferred_element_type=jnp.float32)
             + jnp.dot(x[...], wr[...], preferred_element_type=jnp.float32)
             + b[...])
        o[...] = jnp.maximum(r, 0.0) if relu else r

    grid = (N_PAD // TB,)
    return pl.pallas_call(
        body,
        grid=grid,
        in_specs=[
            pl.BlockSpec((1, TB, D), lambda i: (0, i, 0)),
            pl.BlockSpec((1, TB, D), lambda i: (1, i, 0)),
            pl.BlockSpec((1, TB, 1), lambda i: (0, i, 0)),
            pl.BlockSpec((1, TB, 1), lambda i: (1, i, 0)),
            pl.BlockSpec((TB, D), lambda i: (i, 0)),
            pl.BlockSpec((D, D), lambda i: (0, 0)),
            pl.BlockSpec((D, D), lambda i: (0, 0)),
            pl.BlockSpec((1, D), lambda i: (0, 0)),
        ],
        out_specs=pl.BlockSpec((TB, D), lambda i: (i, 0)),
        out_shape=jax.ShapeDtypeStruct((N_PAD, D), jnp.float32),
    )


_sc_agg_cnt = _sc_aggregate(with_cnt=True)
_sc_agg = _sc_aggregate(with_cnt=False)
_tc_layer1 = _tc_layer(relu=True)
_tc_layer2 = _tc_layer(relu=False)


def kernel(x, edge_index, W1l, b1, W1r, W2l, b2, W2r):
    xp = jnp.pad(x.astype(jnp.float32), ((0, N_PAD - N), (0, 0)))
    src = edge_index[0].astype(jnp.int32).reshape(NW, E // NW)
    dst = edge_index[1].astype(jnp.int32).reshape(NW, E // NW)
    padw = EPW - E // NW
    pad_src = jnp.broadcast_to(jnp.arange(padw, dtype=jnp.int32) % N,
                               (NW, padw))
    pad_dst = jnp.broadcast_to(
        N + jnp.arange(padw, dtype=jnp.int32) % (N_PAD - N), (NW, padw))
    src = jnp.concatenate([src, pad_src], axis=1).reshape(NW, NCH, CH)
    dst = jnp.concatenate([dst, pad_dst], axis=1).reshape(NW, NCH, CH)
    z2 = jnp.zeros((RPT, D), jnp.float32)
    z1 = jnp.zeros((RPT,), jnp.float32)
    o1 = jnp.ones((CH,), jnp.float32)

    agg1, cnt = _sc_agg_cnt(xp, src, dst, z2, z1, o1)
    cntc = cnt.reshape(NC, N_PAD, 1)
    w1l_t = W1l.T
    w1r_t = W1r.T
    h = _tc_layer1(agg1, agg1, cntc, cntc, xp, w1l_t, w1r_t,
                   b1.reshape(1, D))
    (agg2,) = _sc_agg(h, src, dst, z2, z1, o1)
    out = _tc_layer2(agg2, agg2, cntc, cntc, h, W2l.T, W2r.T,
                     b2.reshape(1, D))
    return out[:N]


# speedup vs baseline: 2.8805x; 1.0031x over previous
import functools

import jax
import jax.numpy as jnp
from jax import lax
from jax.experimental import pallas as pl
from jax.experimental.pallas import tpu as pltpu
from jax.experimental.pallas import tpu_sc as plsc

N = 10000
E = 320000
D = 128
NC = 2
NS = 16
NW = NC * NS
N_PAD = 10240
CH = 128
NPH = 2
PHR = 40
NCH = NPH * PHR
EPW = NCH * CH
RPT = N_PAD // NS
TB = 1024


def _sc_aggregate(with_cnt: bool):
    mesh = plsc.VectorSubcoreMesh(core_axis_name="c", subcore_axis_name="s")
    out_type = [jax.ShapeDtypeStruct((NC, N_PAD, D), jnp.float32)]
    scratch = [
        pltpu.VMEM((PHR, CH), jnp.int32),
        pltpu.VMEM((PHR, CH), jnp.int32),
        pltpu.VMEM((2, CH, D), jnp.float32),
        pltpu.VMEM_SHARED((N_PAD, D), jnp.float32),
        pltpu.SemaphoreType.DMA,
        pltpu.SemaphoreType.DMA,
    ]
    if with_cnt:
        out_type.append(jax.ShapeDtypeStruct((NC, N_PAD), jnp.float32))
        scratch += [
            pltpu.VMEM((CH,), jnp.float32),
            pltpu.VMEM_SHARED((N_PAD,), jnp.float32),
            pltpu.SemaphoreType.DMA,
        ]

    def body(x_hbm, src_hbm, dst_hbm, z2_hbm, z1_hbm, o1_hbm,
             out_agg, *rest):
        if with_cnt:
            (out_cnt, src_v, dst_v, rows_v, acc_sh, sem_g, sem_s,
             ones_v, cnt_sh, sem_c) = rest
        else:
            src_v, dst_v, rows_v, acc_sh, sem_g, sem_s = rest
        cid = lax.axis_index("c")
        sid = lax.axis_index("s")
        wid = cid * NS + sid
        pltpu.sync_copy(z2_hbm, acc_sh.at[pl.ds(sid * RPT, RPT)])
        if with_cnt:
            pltpu.sync_copy(z1_hbm, cnt_sh.at[pl.ds(sid * RPT, RPT)])
            pltpu.sync_copy(o1_hbm, ones_v)
        plsc.subcore_barrier()

        for p in range(NPH):
            pltpu.sync_copy(src_hbm.at[wid, pl.ds(p * PHR, PHR)], src_v)
            pltpu.sync_copy(dst_hbm.at[wid, pl.ds(p * PHR, PHR)], dst_v)
            pltpu.async_copy(x_hbm.at[src_v.at[0]], rows_v.at[0], sem_g)

            def step(j, carry):
                b = lax.rem(j, 2)
                nb = lax.rem(j + 1, 2)
                pltpu.make_async_copy(x_hbm.at[src_v.at[j]],
                                      rows_v.at[b], sem_g).wait()
                pltpu.async_copy(rows_v.at[b], acc_sh.at[dst_v.at[j]],
                                 sem_s, add=True)

                @pl.when(j + 1 < PHR)
                def _():
                    pltpu.async_copy(x_hbm.at[src_v.at[j + 1]],
                                     rows_v.at[nb], sem_g)

                if with_cnt:
                    pltpu.async_copy(ones_v, cnt_sh.at[dst_v.at[j]],
                                     sem_c, add=True)
                return carry

            lax.fori_loop(0, PHR, step, 0)
            pltpu.make_async_copy(x_hbm.at[pl.ds(0, PHR * CH)],
                                  acc_sh.at[pl.ds(0, PHR * CH)],
                                  sem_s).wait()
            if with_cnt:
                pltpu.make_async_copy(dst_hbm.at[wid, pl.ds(0, PHR)],
                                      src_v, sem_c).wait()
        plsc.subcore_barrier()
        pltpu.sync_copy(acc_sh.at[pl.ds(sid * RPT, RPT)],
                        out_agg.at[cid, pl.ds(sid * RPT, RPT)])
        if with_cnt:
            pltpu.sync_copy(cnt_sh.at[pl.ds(sid * RPT, RPT)],
                            out_cnt.at[cid, pl.ds(sid * RPT, RPT)])

    return pl.kernel(body, out_type=out_type, mesh=mesh,
                     scratch_types=scratch)


def _tc_layer(relu: bool):
    def body(a0, a1, c0, c1, x, wl, wr, b, o):
        asum = a0[0] + a1[0]
        cs = jnp.maximum(c0[0] + c1[0], 1.0)
        mean = asum / cs
        r = (jnp.dot(mean, wl[...], preferred_element_type=jnp.float32)
             + jnp.dot(x[...], wr[...], preferred_element_type=jnp.float32)
             + b[...])
        o[...] = jnp.maximum(r, 0.0) if relu else r

    grid = (N_PAD // TB,)
    return pl.pallas_call(
        body,
        grid=grid,
        in_specs=[
            pl.BlockSpec((1, TB, D), lambda i: (0, i, 0)),
            pl.BlockSpec((1, TB, D), lambda i: (1, i, 0)),
            pl.BlockSpec((1, TB, 1), lambda i: (0, i, 0)),
            pl.BlockSpec((1, TB, 1), lambda i: (1, i, 0)),
            pl.BlockSpec((TB, D), lambda i: (i, 0)),
            pl.BlockSpec((D, D), lambda i: (0, 0)),
            pl.BlockSpec((D, D), lambda i: (0, 0)),
            pl.BlockSpec((1, D), lambda i: (0, 0)),
        ],
        out_specs=pl.BlockSpec((TB, D), lambda i: (i, 0)),
        out_shape=jax.ShapeDtypeStruct((N_PAD, D), jnp.float32),
    )


_sc_agg_cnt = _sc_aggregate(with_cnt=True)
_sc_agg = _sc_aggregate(with_cnt=False)
_tc_layer1 = _tc_layer(relu=True)
_tc_layer2 = _tc_layer(relu=False)


def kernel(x, edge_index, W1l, b1, W1r, W2l, b2, W2r):
    xp = jnp.pad(x.astype(jnp.float32), ((0, N_PAD - N), (0, 0)))
    src = edge_index[0].astype(jnp.int32).reshape(NW, E // NW)
    dst = edge_index[1].astype(jnp.int32).reshape(NW, E // NW)
    padw = EPW - E // NW
    pad_src = jnp.broadcast_to(jnp.arange(padw, dtype=jnp.int32) % N,
                               (NW, padw))
    pad_dst = jnp.broadcast_to(
        N + jnp.arange(padw, dtype=jnp.int32) % (N_PAD - N), (NW, padw))
    src = jnp.concatenate([src, pad_src], axis=1).reshape(NW, NCH, CH)
    dst = jnp.concatenate([dst, pad_dst], axis=1).reshape(NW, NCH, CH)
    z2 = jnp.zeros((RPT, D), jnp.float32)
    z1 = jnp.zeros((RPT,), jnp.float32)
    o1 = jnp.ones((CH,), jnp.float32)

    agg1, cnt = _sc_agg_cnt(xp, src, dst, z2, z1, o1)
    cntc = cnt.reshape(NC, N_PAD, 1)
    w1l_t = W1l.T
    w1r_t = W1r.T
    h = _tc_layer1(agg1, agg1, cntc, cntc, xp, w1l_t, w1r_t,
                   b1.reshape(1, D))
    (agg2,) = _sc_agg(h, src, dst, z2, z1, o1)
    out = _tc_layer2(agg2, agg2, cntc, cntc, h, W2l.T, W2r.T,
                     b2.reshape(1, D))
    return out[:N]
